# baseline (speedup 1.0000x reference)
import functools

import jax
import jax.numpy as jnp
from jax import lax
from jax.experimental import pallas as pl
from jax.experimental.pallas import tpu as pltpu
from jax.experimental.pallas import tpu_sc as plsc

N_ROWS = 1000000
EMBED_DIM = 16
BATCH = 16384
HIST = 50
B_TOTAL = BATCH * HIST

NC = 2
NS = 16
NW = NC * NS
B_PER_W = B_TOTAL // NW
CHUNK = 3200
N_CHUNKS = B_PER_W // CHUNK


def _body(x_hbm, w_hbm, out_hbm, idx_v, rows_v, sem_g, sem_w):
    wid = lax.axis_index("s") * NC + lax.axis_index("c")
    base = wid * B_PER_W
    pltpu.sync_copy(x_hbm.at[wid], idx_v)
    for g in range(N_CHUNKS):
        pltpu.async_copy(w_hbm.at[idx_v.at[g]], rows_v, sem_g).wait()
        pltpu.sync_copy(rows_v, out_hbm.at[pl.ds(base + g * CHUNK, CHUNK)])


@jax.jit
def _lookup(x_flat, W):
    k = pl.kernel(
        _body,
        out_type=jax.ShapeDtypeStruct((B_TOTAL, EMBED_DIM), jnp.float32),
        mesh=plsc.VectorSubcoreMesh(core_axis_name="c", subcore_axis_name="s"),
        scratch_types=[
            pltpu.VMEM((N_CHUNKS, CHUNK), jnp.int32),
            pltpu.VMEM((CHUNK, EMBED_DIM), jnp.float32),
            pltpu.SemaphoreType.DMA,
            pltpu.SemaphoreType.DMA,
        ],
        compiler_params=pltpu.CompilerParams(use_tc_tiling_on_sc=False),
    )
    return k(x_flat, W)


def kernel(x, W):
    x_flat = x.reshape(NW, N_CHUNKS, CHUNK)
    out = _lookup(x_flat, W)
    return out.reshape(BATCH, HIST, EMBED_DIM)


# speedup vs baseline: 1.0011x; 1.0011x over previous
import functools

import jax
import jax.numpy as jnp
from jax import lax
from jax.experimental import pallas as pl
from jax.experimental.pallas import tpu as pltpu
from jax.experimental.pallas import tpu_sc as plsc

N_ROWS = 1000000
EMBED_DIM = 16
BATCH = 16384
HIST = 50
B_TOTAL = BATCH * HIST

NC = 2
NS = 16
NW = NC * NS
B_PER_W = B_TOTAL // NW
CHUNK = 3200
N_CHUNKS = B_PER_W // CHUNK


def _body(x_hbm, w_hbm, out_hbm, idx_v, rows_v, sem_g, sem_w):
    wid = lax.axis_index("s") * NC + lax.axis_index("c")
    base = wid * B_PER_W
    pltpu.sync_copy(x_hbm.at[wid], idx_v)
    gathers = [pltpu.async_copy(w_hbm.at[idx_v.at[0]], rows_v.at[0], sem_g)]
    wbs = []
    for g in range(N_CHUNKS):
        gathers[g].wait()
        if g + 1 < N_CHUNKS:
            if g >= 1:
                wbs[g - 1].wait()
            gathers.append(
                pltpu.async_copy(
                    w_hbm.at[idx_v.at[g + 1]], rows_v.at[(g + 1) & 1], sem_g
                )
            )
        wbs.append(
            pltpu.async_copy(
                rows_v.at[g & 1],
                out_hbm.at[pl.ds(base + g * CHUNK, CHUNK)],
                sem_w,
            )
        )
    wbs[N_CHUNKS - 2].wait()
    wbs[N_CHUNKS - 1].wait()


@jax.jit
def _lookup(x_flat, W):
    k = pl.kernel(
        _body,
        out_type=jax.ShapeDtypeStruct((B_TOTAL, EMBED_DIM), jnp.float32),
        mesh=plsc.VectorSubcoreMesh(core_axis_name="c", subcore_axis_name="s"),
        scratch_types=[
            pltpu.VMEM((N_CHUNKS, CHUNK), jnp.int32),
            pltpu.VMEM((2, CHUNK, EMBED_DIM), jnp.float32),
            pltpu.SemaphoreType.DMA,
            pltpu.SemaphoreType.DMA,
        ],
        compiler_params=pltpu.CompilerParams(use_tc_tiling_on_sc=False),
    )
    return k(x_flat, W)


def kernel(x, W):
    x_flat = x.reshape(NW, N_CHUNKS, CHUNK)
    out = _lookup(x_flat, W)
    return out.reshape(BATCH, HIST, EMBED_DIM)


# speedup vs baseline: 1.4931x; 1.4914x over previous
import functools

import jax
import jax.numpy as jnp
from jax import lax
from jax.experimental import pallas as pl
from jax.experimental.pallas import tpu as pltpu
from jax.experimental.pallas import tpu_sc as plsc

N_ROWS = 1000000
EMBED_DIM = 16
BATCH = 16384
HIST = 50

NC = 2
NS = 16
NW = NC * NS
TJ = BATCH // 128
TJ_PER_W = TJ // NW
BW = 128 * TJ_PER_W


def _body(xT_hbm, w_hbm, out_hbm, idx_v, rows_v, tbuf, sem_g, sem_o):
    wid = lax.axis_index("s") * NC + lax.axis_index("c")
    d_iota = lax.iota(jnp.int32, EMBED_DIM)
    ti_idx = d_iota // 8
    r_idx = d_iota % 8
    zeros = jnp.zeros((EMBED_DIM,), jnp.int32)

    def per_h(h, carry):
        pltpu.sync_copy(xT_hbm.at[h, pl.ds(wid * BW, BW)], idx_v)
        pltpu.async_copy(w_hbm.at[idx_v], rows_v, sem_g).wait()

        def per_c(c, carry2):
            vals = rows_v[c]
            j_splat = zeros + (c // 128)
            c_splat = zeros + (c % 128)
            plsc.store_scatter(tbuf, [ti_idx, j_splat, r_idx, c_splat], vals)
            return carry2

        lax.fori_loop(0, BW, per_c, 0)

        cp0 = pltpu.async_copy(
            tbuf.at[0], out_hbm.at[h, 0, pl.ds(wid * TJ_PER_W, TJ_PER_W)], sem_o
        )
        cp1 = pltpu.async_copy(
            tbuf.at[1], out_hbm.at[h, 1, pl.ds(wid * TJ_PER_W, TJ_PER_W)], sem_o
        )
        cp0.wait()
        cp1.wait()
        return carry

    lax.fori_loop(0, HIST, per_h, 0)


@jax.jit
def _lookup(xT, W):
    k = pl.kernel(
        _body,
        out_type=jax.ShapeDtypeStruct((HIST, 2, TJ, 8, 128), jnp.float32),
        mesh=plsc.VectorSubcoreMesh(core_axis_name="c", subcore_axis_name="s"),
        scratch_types=[
            pltpu.VMEM((BW,), jnp.int32),
            pltpu.VMEM((BW, EMBED_DIM), jnp.float32),
            pltpu.VMEM((2, TJ_PER_W, 8, 128), jnp.float32),
            pltpu.SemaphoreType.DMA,
            pltpu.SemaphoreType.DMA,
        ],
        compiler_params=pltpu.CompilerParams(
            use_tc_tiling_on_sc=False, needs_layout_passes=False
        ),
    )
    return k(xT, W)


def kernel(x, W):
    out5 = _lookup(x.T, W)
    return out5.transpose(2, 4, 0, 1, 3).reshape(BATCH, HIST, EMBED_DIM)


# speedup vs baseline: 1.6548x; 1.1083x over previous
import functools

import jax
import jax.numpy as jnp
from jax import lax
from jax.experimental import pallas as pl
from jax.experimental.pallas import tpu as pltpu
from jax.experimental.pallas import tpu_sc as plsc

N_ROWS = 1000000
EMBED_DIM = 16
BATCH = 16384
HIST = 50

NC = 2
NS = 16
NW = NC * NS
TJ = BATCH // 128
TJ_PER_W = TJ // NW
BW = 128 * TJ_PER_W


def _body(
    xT_hbm, w_hbm, out_hbm,
    idx_a, idx_b, rows_a, rows_b, tbuf_a, tbuf_b,
    sem_g, sem_o,
):
    wid = lax.axis_index("s") * NC + lax.axis_index("c")
    base = wid * BW
    w16 = w_hbm

    d_iota = lax.iota(jnp.int32, EMBED_DIM)
    ti_idx = d_iota // 8
    r_idx = d_iota % 8
    zeros = jnp.zeros((EMBED_DIM,), jnp.int32)

    def stage(h, idx_v, rows_v):
        pltpu.sync_copy(xT_hbm.at[h, pl.ds(base, BW)], idx_v)
        return pltpu.async_copy(w16.at[idx_v], rows_v, sem_g)

    def transpose(rows_v, tbuf):
        for j in range(TJ_PER_W):
            j_splat = zeros + j

            def per_c(c, carry):
                vals = rows_v[j * 128 + c]
                plsc.store_scatter(
                    tbuf, [ti_idx, j_splat, r_idx, zeros + c], vals
                )
                return carry

            lax.fori_loop(0, 128, per_c, 0, unroll=8)

    def write_out(h, tbuf):
        pltpu.async_copy(
            tbuf.at[0], out_hbm.at[h, 0, pl.ds(wid * TJ_PER_W, TJ_PER_W)], sem_o
        )
        pltpu.async_copy(
            tbuf.at[1], out_hbm.at[h, 1, pl.ds(wid * TJ_PER_W, TJ_PER_W)], sem_o
        )

    def drain_writes(h, tbuf):
        pltpu.make_async_copy(
            tbuf.at[0], out_hbm.at[h, 0, pl.ds(wid * TJ_PER_W, TJ_PER_W)], sem_o
        ).wait()
        pltpu.make_async_copy(
            tbuf.at[1], out_hbm.at[h, 1, pl.ds(wid * TJ_PER_W, TJ_PER_W)], sem_o
        ).wait()

    g0 = stage(0, idx_a, rows_a)

    def step(k, carry):
        h0 = 2 * k
        h1 = 2 * k + 1
        gb = stage(h1, idx_b, rows_b)
        gb.wait()

        @pl.when(k > 0)
        def _():
            drain_writes(h0, tbuf_a)

        transpose(rows_a, tbuf_a)
        write_out(h0, tbuf_a)

        ga = stage(jnp.minimum(h1 + 1, HIST - 1), idx_a, rows_a)
        ga.wait()

        @pl.when(k > 0)
        def _():
            drain_writes(h1, tbuf_b)

        transpose(rows_b, tbuf_b)
        write_out(h1, tbuf_b)
        return carry

    lax.fori_loop(0, HIST // 2, step, 0)

    pltpu.make_async_copy(w16.at[idx_a], rows_a, sem_g).wait()
    drain_writes(HIST - 2, tbuf_a)
    drain_writes(HIST - 1, tbuf_b)


@jax.jit
def _lookup(xT, Wr):
    k = pl.kernel(
        _body,
        out_type=jax.ShapeDtypeStruct((HIST, 2, TJ, 8, 128), jnp.float32),
        mesh=plsc.VectorSubcoreMesh(core_axis_name="c", subcore_axis_name="s"),
        scratch_types=[
            pltpu.VMEM((BW,), jnp.int32),
            pltpu.VMEM((BW,), jnp.int32),
            pltpu.VMEM((BW, EMBED_DIM), jnp.float32),
            pltpu.VMEM((BW, EMBED_DIM), jnp.float32),
            pltpu.VMEM((2, TJ_PER_W, 8, 128), jnp.float32),
            pltpu.VMEM((2, TJ_PER_W, 8, 128), jnp.float32),
            pltpu.SemaphoreType.DMA,
            pltpu.SemaphoreType.DMA,
        ],
        compiler_params=pltpu.CompilerParams(
            use_tc_tiling_on_sc=False, needs_layout_passes=False
        ),
    )
    return k(xT, Wr)


def kernel(x, W):
    out5 = _lookup(x.T, W)
    return out5.transpose(2, 4, 0, 1, 3).reshape(BATCH, HIST, EMBED_DIM)
